# Initial kernel scaffold; baseline (speedup 1.0000x reference)
import jax
import jax.numpy as jnp
from jax.experimental import pallas as pl


def kernel(x, W_bot0, b_bot0, W_bot1, b_bot1, W_bot2, b_bot2, embedding_table, W_top0, b_top0, W_top1, b_top1, W_top2, b_top2, W_top3, b_top3, W_out, b_out):
    raise NotImplementedError("write your pallas kernel here")


# speedup vs baseline: 13.3750x; 13.3750x over previous
import functools

import jax
import jax.numpy as jnp
from jax import lax
from jax.experimental import pallas as pl
from jax.experimental.pallas import tpu as pltpu
from jax.experimental.pallas import tpu_sc as plsc

VOCAB = 1000000
EMBED = 128
BATCH = 4096
NUM_DENSE = 13
NUM_CAT = 26

N_IDX = BATCH * NUM_CAT
NC, NS = 2, 16
NW = NC * NS
PER_W = N_IDX // NW
CHUNK = 128
N_CHUNK = PER_W // CHUNK


def _gather_body(table_hbm, idx_hbm, out_hbm, idx_v, buf0, buf1, gsem0, gsem1,
                 ssem0, ssem1):
    wid = lax.axis_index("s") * NC + lax.axis_index("c")
    pltpu.sync_copy(idx_hbm.at[pl.ds(wid * PER_W, PER_W)], idx_v)

    bufs = (buf0, buf1)
    gsems = (gsem0, gsem1)
    ssems = (ssem0, ssem1)
    out_base = wid * PER_W

    def gather(c, b):
        return pltpu.make_async_copy(
            table_hbm.at[idx_v.at[pl.ds(c * CHUNK, CHUNK)]], bufs[b], gsems[b])

    def store(c, b):
        return pltpu.make_async_copy(
            bufs[b], out_hbm.at[pl.ds(out_base + c * CHUNK, CHUNK)], ssems[b])

    gather(0, 0).start()
    for c in range(N_CHUNK):
        b = c % 2
        if c + 1 < N_CHUNK:
            nb = (c + 1) % 2
            if c >= 1:
                store(c - 1, nb).wait()
            gather(c + 1, nb).start()
        gather(c, b).wait()
        store(c, b).start()
    store(N_CHUNK - 2, (N_CHUNK - 2) % 2).wait()
    store(N_CHUNK - 1, (N_CHUNK - 1) % 2).wait()


def _sc_gather(table, idx_flat):
    mesh = plsc.VectorSubcoreMesh(core_axis_name="c", subcore_axis_name="s")
    return pl.kernel(
        _gather_body,
        out_type=jax.ShapeDtypeStruct((N_IDX, EMBED), jnp.float32),
        mesh=mesh,
        scratch_types=[
            pltpu.VMEM((PER_W,), jnp.int32),
            pltpu.VMEM((CHUNK, EMBED), jnp.float32),
            pltpu.VMEM((CHUNK, EMBED), jnp.float32),
            pltpu.SemaphoreType.DMA,
            pltpu.SemaphoreType.DMA,
            pltpu.SemaphoreType.DMA,
            pltpu.SemaphoreType.DMA,
        ],
    )(table, idx_flat)


def _mlp_body(x_ref, emb_ref, wb0, bb0, wb1, bb1, wb2, bb2,
              wt0a, wt0b, bt0, wt1, bt1, wt2, bt2, wt3, bt3, wo, bo,
              out_ref):
    f32 = jnp.float32
    xb = x_ref[:, :NUM_DENSE]
    bot = jax.nn.relu(jnp.dot(xb, wb0[:], preferred_element_type=f32) + bb0[:])
    bot = bot + jax.nn.relu(jnp.dot(bot, wb1[:], preferred_element_type=f32) + bb1[:])
    bot = bot + jax.nn.relu(jnp.dot(bot, wb2[:], preferred_element_type=f32) + bb2[:])
    t = jax.nn.relu(
        jnp.dot(bot, wt0a[:], preferred_element_type=f32)
        + jnp.dot(emb_ref[:], wt0b[:], preferred_element_type=f32)
        + bt0[:])
    t = t + jax.nn.relu(jnp.dot(t, wt1[:], preferred_element_type=f32) + bt1[:])
    t = t + jax.nn.relu(jnp.dot(t, wt2[:], preferred_element_type=f32) + bt2[:])
    t = t + jax.nn.relu(jnp.dot(t, wt3[:], preferred_element_type=f32) + bt3[:])
    out_ref[:] = jnp.dot(t, wo[:], preferred_element_type=f32) + bo[:]


_BB = 512


def _tc_mlp(x, emb, wb0, bb0, wb1, bb1, wb2, bb2,
            wt0a, wt0b, bt0, wt1, bt1, wt2, bt2, wt3, bt3, wo, bo):
    grid = (BATCH // _BB,)

    def bspec(shape):
        return pl.BlockSpec(shape, lambda i: (0,) * len(shape))

    return pl.pallas_call(
        _mlp_body,
        grid=grid,
        in_specs=[
            pl.BlockSpec((_BB, NUM_DENSE + NUM_CAT), lambda i: (i, 0)),
            pl.BlockSpec((_BB, NUM_CAT * EMBED), lambda i: (i, 0)),
            bspec(wb0.shape), bspec(bb0.shape),
            bspec(wb1.shape), bspec(bb1.shape),
            bspec(wb2.shape), bspec(bb2.shape),
            bspec(wt0a.shape), bspec(wt0b.shape), bspec(bt0.shape),
            bspec(wt1.shape), bspec(bt1.shape),
            bspec(wt2.shape), bspec(bt2.shape),
            bspec(wt3.shape), bspec(bt3.shape),
            bspec(wo.shape), bspec(bo.shape),
        ],
        out_specs=pl.BlockSpec((_BB, 1), lambda i: (i, 0)),
        out_shape=jax.ShapeDtypeStruct((BATCH, 1), jnp.float32),
    )(x, emb, wb0, bb0, wb1, bb1, wb2, bb2,
      wt0a, wt0b, bt0, wt1, bt1, wt2, bt2, wt3, bt3, wo, bo)


def kernel(x, W_bot0, b_bot0, W_bot1, b_bot1, W_bot2, b_bot2, embedding_table,
           W_top0, b_top0, W_top1, b_top1, W_top2, b_top2, W_top3, b_top3,
           W_out, b_out):
    idx = jnp.asarray(x[:, NUM_DENSE:], jnp.int32) % VOCAB
    emb = _sc_gather(embedding_table, idx.reshape(-1))
    emb = emb.reshape(BATCH, NUM_CAT * EMBED)
    wt0a = W_top0[:256]
    wt0b = W_top0[256:]
    return _tc_mlp(
        x, emb,
        W_bot0, b_bot0.reshape(1, -1),
        W_bot1, b_bot1.reshape(1, -1),
        W_bot2, b_bot2.reshape(1, -1),
        wt0a, wt0b, b_top0.reshape(1, -1),
        W_top1, b_top1.reshape(1, -1),
        W_top2, b_top2.reshape(1, -1),
        W_top3, b_top3.reshape(1, -1),
        W_out, b_out.reshape(1, -1))
